# baseline (speedup 1.0000x reference)
import functools
import math

import jax
import jax.numpy as jnp
from jax import lax
from jax.experimental import pallas as pl
from jax.experimental.pallas import tpu as pltpu
from jax.experimental.pallas import tpu_sc as plsc

_EMB = 32
_B = 16384
_L = 50
_NW = 32
_BW = _B // _NW
_CHUNK = 128
_NCH = _BW // _CHUNK
_SCALE = math.sqrt(_EMB)

_mesh = plsc.VectorSubcoreMesh(core_axis_name="c", subcore_axis_name="s")


@functools.partial(
    pl.kernel,
    mesh=_mesh,
    compiler_params=pltpu.CompilerParams(
        use_tc_tiling_on_sc=False, needs_layout_passes=False),
    out_type=jax.ShapeDtypeStruct((_L, _EMB // 8, (_B // 128) * 1024), jnp.float32),
    scratch_types=[
        pltpu.VMEM((_L, _NCH, _CHUNK), jnp.int32),
        pltpu.VMEM((_BW, _EMB), jnp.float32),
        pltpu.VMEM((_BW, _EMB), jnp.float32),
        pltpu.VMEM((_BW, _EMB), jnp.float32),
        pltpu.VMEM((_EMB * _BW,), jnp.float32),
        pltpu.VMEM((_EMB * _BW,), jnp.float32),
        pltpu.SemaphoreType.DMA,
        pltpu.SemaphoreType.DMA,
        pltpu.SemaphoreType.DMA,
        pltpu.SemaphoreType.DMA,
        pltpu.SemaphoreType.DMA,
    ],
)
def _emb_lookup(tokt_hbm, table_hbm, out_hbm, tok_v, gb0, gb1, gb2,
                tb0, tb1, g0, g1, g2, o0, o1):
    wid = lax.axis_index("s") * 2 + lax.axis_index("c")
    base = wid * _BW
    for k in range(_NCH):
        pltpu.sync_copy(
            tokt_hbm.at[:, pl.ds(base + k * _CHUNK, _CHUNK)],
            tok_v.at[:, k],
        )

    gbufs = ((gb0, g0), (gb1, g1), (gb2, g2))
    tbufs = ((tb0, o0), (tb1, o1))
    _lane = lax.iota(jnp.int32, 16)
    lanepat = (_lane >> 3) * 4096 + (_lane & 7) * 128

    def issue_g(l, gi):
        gbuf, gsem = gbufs[gi]
        for k in range(_NCH):
            pltpu.make_async_copy(
                table_hbm.at[tok_v.at[l, k]],
                gbuf.at[pl.ds(k * _CHUNK, _CHUNK)],
                gsem,
            ).start()

    def wait_g(gi):
        gbuf, gsem = gbufs[gi]
        for k in range(_NCH):
            pltpu.make_async_copy(
                table_hbm.at[tok_v.at[0, k]],
                gbuf.at[pl.ds(k * _CHUNK, _CHUNK)],
                gsem,
            ).wait()

    def issue_o(l, ti):
        tbuf, osem = tbufs[ti]
        for et in range(_EMB // 8):
            pltpu.make_async_copy(
                tbuf.at[pl.ds(et * 4096, 4096)],
                out_hbm.at[l, et, pl.ds(wid * 4096, 4096)],
                osem,
            ).start()

    def wait_o(ti):
        tbuf, osem = tbufs[ti]
        for et in range(_EMB // 8):
            pltpu.make_async_copy(
                tbuf.at[pl.ds(et * 4096, 4096)],
                out_hbm.at[0, et, pl.ds(wid * 4096, 4096)],
                osem,
            ).wait()

    def transpose_scale(gi, ti):
        gbuf, _ = gbufs[gi]
        tbuf, _ = tbufs[ti]

        def body(i, c):
            for r in range(4):
                row = i * 4 + r
                idx0 = lanepat + ((row >> 7) * 1024 + (row & 127))
                v0 = gbuf[row, pl.ds(0, 16)] * _SCALE
                plsc.store_scatter(tbuf, [idx0], v0)
                v1 = gbuf[row, pl.ds(16, 16)] * _SCALE
                plsc.store_scatter(tbuf, [idx0 + 8192], v1)
            return c
        lax.fori_loop(0, _BW // 4, body, 0)

    def stage(l, gi, ti, prefetch, first):
        if prefetch:
            issue_g(l + 2, (gi + 2) % 3)
        wait_g(gi)
        if not first:
            wait_o(ti)
        transpose_scale(gi, ti)
        issue_o(l, ti)

    issue_g(0, 0)
    issue_g(1, 1)
    stage(0, 0, 0, True, True)
    stage(1, 1, 1, True, True)

    def six(t, c):
        for j in range(6):
            stage(t * 6 + 2 + j, (2 + j) % 3, j % 2, True, False)
        return c
    lax.fori_loop(0, 7, six, 0)

    for l in range(_L - 6, _L):
        stage(l, l % 3, l % 2, l + 2 < _L, False)
    wait_o(0)
    wait_o(1)


def kernel(tokens, table):
    out = _emb_lookup(tokens.T.astype(jnp.int32), table)
    out = out.reshape(_L, _EMB // 8, _B // 128, 8, 128)
    return out.transpose(2, 4, 0, 1, 3).reshape(_B, _L, _EMB)


# speedup vs baseline: 1.8564x; 1.8564x over previous
import functools
import math

import jax
import jax.numpy as jnp
from jax import lax
from jax.experimental import pallas as pl
from jax.experimental.pallas import tpu as pltpu
from jax.experimental.pallas import tpu_sc as plsc

_EMB = 32
_B = 16384
_L = 50
_NW = 32
_BW = _B // _NW
_CHUNK = 128
_NCH = _BW // _CHUNK
_SCALE = math.sqrt(_EMB)

_mesh = plsc.VectorSubcoreMesh(core_axis_name="c", subcore_axis_name="s")


@functools.partial(
    pl.kernel,
    mesh=_mesh,
    compiler_params=pltpu.CompilerParams(
        use_tc_tiling_on_sc=False, needs_layout_passes=False),
    out_type=jax.ShapeDtypeStruct((_L, _EMB // 8, (_B // 128) * 1024), jnp.float32),
    scratch_types=[
        pltpu.VMEM((_L, _NCH, _CHUNK), jnp.int32),
        pltpu.VMEM((_BW, _EMB), jnp.float32),
        pltpu.VMEM((_BW, _EMB), jnp.float32),
        pltpu.VMEM((_BW, _EMB), jnp.float32),
        pltpu.VMEM((_EMB * _BW,), jnp.float32),
        pltpu.VMEM((_EMB * _BW,), jnp.float32),
        pltpu.SemaphoreType.DMA,
        pltpu.SemaphoreType.DMA,
        pltpu.SemaphoreType.DMA,
        pltpu.SemaphoreType.DMA,
        pltpu.SemaphoreType.DMA,
    ],
)
def _emb_lookup(tokt_hbm, table_hbm, out_hbm, tok_v, gb0, gb1, gb2,
                tb0, tb1, g0, g1, g2, o0, o1):
    wid = lax.axis_index("s") * 2 + lax.axis_index("c")
    base = wid * _BW
    for k in range(_NCH):
        pltpu.sync_copy(
            tokt_hbm.at[:, pl.ds(base + k * _CHUNK, _CHUNK)],
            tok_v.at[:, k],
        )

    gbufs = ((gb0, g0), (gb1, g1), (gb2, g2))
    tbufs = ((tb0, o0), (tb1, o1))
    _lane = lax.iota(jnp.int32, 16)
    lanepat = (_lane >> 3) * 4096 + (_lane & 7) * 128

    def issue_g(l, gi):
        gbuf, gsem = gbufs[gi]
        for k in range(_NCH):
            pltpu.make_async_copy(
                table_hbm.at[tok_v.at[l, k]],
                gbuf.at[pl.ds(k * _CHUNK, _CHUNK)],
                gsem,
            ).start()

    def wait_g(gi):
        gbuf, gsem = gbufs[gi]
        for k in range(_NCH):
            pltpu.make_async_copy(
                table_hbm.at[tok_v.at[0, k]],
                gbuf.at[pl.ds(k * _CHUNK, _CHUNK)],
                gsem,
            ).wait()

    def issue_o(l, ti):
        tbuf, osem = tbufs[ti]
        for et in range(_EMB // 8):
            pltpu.make_async_copy(
                tbuf.at[pl.ds(et * 4096, 4096)],
                out_hbm.at[l, et, pl.ds(wid * 4096, 4096)],
                osem,
            ).start()

    def wait_o(ti):
        tbuf, osem = tbufs[ti]
        for et in range(_EMB // 8):
            pltpu.make_async_copy(
                tbuf.at[pl.ds(et * 4096, 4096)],
                out_hbm.at[0, et, pl.ds(wid * 4096, 4096)],
                osem,
            ).wait()

    def transpose_scale(gi, ti):
        gbuf, _ = gbufs[gi]
        tbuf, _ = tbufs[ti]

        @functools.partial(plsc.parallel_loop, 0, _BW, unroll=8)
        def _(row):
            idx0 = lanepat + ((row >> 7) * 1024 + (row & 127))
            v0 = gbuf[row, pl.ds(0, 16)] * _SCALE
            plsc.store_scatter(tbuf, [idx0], v0)
            v1 = gbuf[row, pl.ds(16, 16)] * _SCALE
            plsc.store_scatter(tbuf, [idx0 + 8192], v1)

    def stage(l, gi, ti, prefetch, first):
        if prefetch:
            issue_g(l + 2, (gi + 2) % 3)
        wait_g(gi)
        if not first:
            wait_o(ti)
        transpose_scale(gi, ti)
        issue_o(l, ti)

    issue_g(0, 0)
    issue_g(1, 1)
    stage(0, 0, 0, True, True)
    stage(1, 1, 1, True, True)

    def six(t, c):
        for j in range(6):
            stage(t * 6 + 2 + j, (2 + j) % 3, j % 2, True, False)
        return c
    lax.fori_loop(0, 7, six, 0)

    for l in range(_L - 6, _L):
        stage(l, l % 3, l % 2, l + 2 < _L, False)
    wait_o(0)
    wait_o(1)


def kernel(tokens, table):
    out = _emb_lookup(tokens.T.astype(jnp.int32), table)
    out = out.reshape(_L, _EMB // 8, _B // 128, 8, 128)
    return out.transpose(2, 4, 0, 1, 3).reshape(_B, _L, _EMB)
